# Initial kernel scaffold; baseline (speedup 1.0000x reference)
import jax
import jax.numpy as jnp
from jax.experimental import pallas as pl


def kernel(x, edge_index, edge_attr, batch, Wq, bq, Wk, bk, Wv, bv, We, be, Wo, bo, gamma, beta):
    raise NotImplementedError("write your pallas kernel here")


# speedup vs baseline: 23.2333x; 23.2333x over previous
import functools

import jax
import jax.numpy as jnp
from jax import lax
from jax.experimental import pallas as pl
from jax.experimental.pallas import tpu as pltpu
from jax.experimental.pallas import tpu_sc as plsc

N = 10000
E = 320000
IN_DIM = 128
OUT_DIM = 128
EDGE_DIM = 16
H = 8
DH = OUT_DIM // H
SCALE = DH ** -0.5
EPS = 1e-5

NC = 2
NS = 16
NW = NC * NS
NPAD = 10240
WROWS = OUT_DIM // NW

CE = 4000
NCH = E // CE
GRP = CE // 16

QKV_BLK = 2000
EDGE_BLK = 2560
NODE_BLK = 2048

_f32 = jnp.float32


def _qkv_body(x_ref, wq_ref, wk_ref, wv_ref, bq_ref, bk_ref, bv_ref,
              q_ref, k_ref, v_ref):
    xb = x_ref[...]
    q_ref[...] = jnp.dot(xb, wq_ref[...], preferred_element_type=_f32) + bq_ref[...]
    k_ref[...] = jnp.dot(xb, wk_ref[...], preferred_element_type=_f32) + bk_ref[...]
    v_ref[...] = jnp.dot(xb, wv_ref[...], preferred_element_type=_f32) + bv_ref[...]


def _qkv(x, Wq, Wk, Wv, bq, bk, bv):
    blk = pl.BlockSpec((QKV_BLK, IN_DIM), lambda i: (i, 0))
    wspec = pl.BlockSpec((IN_DIM, OUT_DIM), lambda i: (0, 0))
    bspec = pl.BlockSpec((1, OUT_DIM), lambda i: (0, 0))
    return pl.pallas_call(
        _qkv_body,
        grid=(N // QKV_BLK,),
        in_specs=[blk, wspec, wspec, wspec, bspec, bspec, bspec],
        out_specs=[pl.BlockSpec((QKV_BLK, OUT_DIM), lambda i: (i, 0))] * 3,
        out_shape=[jax.ShapeDtypeStruct((N, OUT_DIM), _f32)] * 3,
    )(x, Wq, Wk, Wv, bq.reshape(1, -1), bk.reshape(1, -1), bv.reshape(1, -1))


GCH = 128
ET = E // NW
GFULL = ET // GCH
GTAIL = ET - GFULL * GCH


def _gather_body(src_hbm, dst_hbm, q_hbm, k_hbm, v_hbm,
                 qd_hbm, ks_hbm, vs_hbm,
                 sidx, didx, sidx_t, didx_t,
                 qv, kv, vv, qv_t, kv_t, vv_t,
                 sem0, sem1, sem2):
    cid = lax.axis_index("c")
    sid = lax.axis_index("s")
    wid = sid * NC + cid
    base = wid * ET

    def chunk(j, carry):
        off = pl.multiple_of(base + j * GCH, 8)
        pltpu.sync_copy(src_hbm.at[pl.ds(off, GCH)], sidx)
        pltpu.sync_copy(dst_hbm.at[pl.ds(off, GCH)], didx)
        cq = pltpu.async_copy(q_hbm.at[didx], qv, sem0)
        ck = pltpu.async_copy(k_hbm.at[sidx], kv, sem1)
        cv = pltpu.async_copy(v_hbm.at[sidx], vv, sem2)
        cq.wait()
        ck.wait()
        cv.wait()
        pltpu.sync_copy(qv, qd_hbm.at[pl.ds(off, GCH)])
        pltpu.sync_copy(kv, ks_hbm.at[pl.ds(off, GCH)])
        pltpu.sync_copy(vv, vs_hbm.at[pl.ds(off, GCH)])
        return carry

    lax.fori_loop(0, GFULL, chunk, 0)
    off = base + GFULL * GCH
    pltpu.sync_copy(src_hbm.at[pl.ds(off, GTAIL)], sidx_t)
    pltpu.sync_copy(dst_hbm.at[pl.ds(off, GTAIL)], didx_t)
    cq = pltpu.async_copy(q_hbm.at[didx_t], qv_t, sem0)
    ck = pltpu.async_copy(k_hbm.at[sidx_t], kv_t, sem1)
    cv = pltpu.async_copy(v_hbm.at[sidx_t], vv_t, sem2)
    cq.wait()
    ck.wait()
    cv.wait()
    pltpu.sync_copy(qv_t, qd_hbm.at[pl.ds(off, GTAIL)])
    pltpu.sync_copy(kv_t, ks_hbm.at[pl.ds(off, GTAIL)])
    pltpu.sync_copy(vv_t, vs_hbm.at[pl.ds(off, GTAIL)])


def _gather(src, dst, q, k, v):
    mesh = plsc.VectorSubcoreMesh(core_axis_name="c", subcore_axis_name="s",
                                  num_cores=NC, num_subcores=NS)
    f = functools.partial(
        pl.kernel,
        mesh=mesh,
        out_type=[jax.ShapeDtypeStruct((E, OUT_DIM), _f32)] * 3,
        scratch_types=[
            pltpu.VMEM((GCH,), jnp.int32),
            pltpu.VMEM((GCH,), jnp.int32),
            pltpu.VMEM((GTAIL,), jnp.int32),
            pltpu.VMEM((GTAIL,), jnp.int32),
            pltpu.VMEM((GCH, OUT_DIM), _f32),
            pltpu.VMEM((GCH, OUT_DIM), _f32),
            pltpu.VMEM((GCH, OUT_DIM), _f32),
            pltpu.VMEM((GTAIL, OUT_DIM), _f32),
            pltpu.VMEM((GTAIL, OUT_DIM), _f32),
            pltpu.VMEM((GTAIL, OUT_DIM), _f32),
            pltpu.SemaphoreType.DMA,
            pltpu.SemaphoreType.DMA,
            pltpu.SemaphoreType.DMA,
        ],
    )(_gather_body)
    return f(src, dst, q, k, v)


def _edge_body(qd_ref, ks_ref, vs_ref, ea_ref, g_ref, wex_ref, bex_ref,
               dsel_ref, eye_ref, wt_ref, et_ref):
    prod = qd_ref[...] * ks_ref[...]
    score = (jnp.dot(prod, g_ref[...], preferred_element_type=_f32)
             + jnp.dot(ea_ref[...], wex_ref[...], preferred_element_type=_f32)
             + bex_ref[...])
    exx = jnp.exp(score)
    w = vs_ref[...] * exx
    wt_ref[...] = lax.dot_general(
        eye_ref[...], w, (((0,), (1,)), ((), ())),
        preferred_element_type=_f32)
    et_ref[...] = lax.dot_general(
        dsel_ref[...], exx, (((1,), (1,)), ((), ())),
        preferred_element_type=_f32)


def _edge_compute(qd, ks, vs, ea, G, WeX, beX, Dsel, Eye):
    eblk = pl.BlockSpec((EDGE_BLK, OUT_DIM), lambda i: (i, 0))
    return pl.pallas_call(
        _edge_body,
        grid=(E // EDGE_BLK,),
        in_specs=[eblk, eblk, eblk,
                  pl.BlockSpec((EDGE_BLK, EDGE_DIM), lambda i: (i, 0)),
                  pl.BlockSpec((OUT_DIM, OUT_DIM), lambda i: (0, 0)),
                  pl.BlockSpec((EDGE_DIM, OUT_DIM), lambda i: (0, 0)),
                  pl.BlockSpec((1, OUT_DIM), lambda i: (0, 0)),
                  pl.BlockSpec((H, OUT_DIM), lambda i: (0, 0)),
                  pl.BlockSpec((OUT_DIM, OUT_DIM), lambda i: (0, 0))],
        out_specs=[pl.BlockSpec((OUT_DIM, EDGE_BLK), lambda i: (0, i)),
                   pl.BlockSpec((H, EDGE_BLK), lambda i: (0, i))],
        out_shape=[jax.ShapeDtypeStruct((OUT_DIM, E), _f32),
                   jax.ShapeDtypeStruct((H, E), _f32)],
    )(qd, ks, vs, ea, G, WeX, beX, Dsel, Eye)


def _scatter_body(wt_hbm, et_hbm, dst_hbm, z_hbm, owt_hbm, oet_hbm,
                  didx, wb, eb, a0, a1, a2, a3, ae):
    cid = lax.axis_index("c")
    sid = lax.axis_index("s")
    u = sid * NC + cid
    accs = [a0, a1, a2, a3]
    for a in accs:
        pltpu.sync_copy(z_hbm, a)
    row0 = u * WROWS

    def chunk(j, carry):
        off = pl.multiple_of(j * CE, 8)
        pltpu.sync_copy(dst_hbm.at[pl.ds(off, CE)], didx)
        for r in range(WROWS):
            pltpu.sync_copy(wt_hbm.at[row0 + r, pl.ds(off, CE)], wb.at[r])

        def grp(g, c2):
            s16 = pl.multiple_of(g * 16, 16)
            idxv = didx[pl.ds(s16, 16)]
            for r in range(WROWS):
                plsc.addupdate_scatter(accs[r], [idxv],
                                       wb[r, pl.ds(s16, 16)])
            return c2

        lax.fori_loop(0, GRP, grp, 0, unroll=4)
        return carry

    lax.fori_loop(0, NCH, chunk, 0)
    for r in range(WROWS):
        pltpu.sync_copy(accs[r], owt_hbm.at[row0 + r])

    @pl.when(u < H)
    def _():
        pltpu.sync_copy(z_hbm, ae)

        def chunk_e(j, carry):
            off = pl.multiple_of(j * CE, 8)
            pltpu.sync_copy(dst_hbm.at[pl.ds(off, CE)], didx)
            pltpu.sync_copy(et_hbm.at[u, pl.ds(off, CE)], eb)

            def grp(g, c2):
                s16 = pl.multiple_of(g * 16, 16)
                idxv = didx[pl.ds(s16, 16)]
                plsc.addupdate_scatter(ae, [idxv], eb[pl.ds(s16, 16)])
                return c2

            lax.fori_loop(0, GRP, grp, 0, unroll=4)
            return carry

        lax.fori_loop(0, NCH, chunk_e, 0)
        pltpu.sync_copy(ae, oet_hbm.at[u])


def _scatter(wT, eT, dst, z):
    mesh = plsc.VectorSubcoreMesh(core_axis_name="c", subcore_axis_name="s",
                                  num_cores=NC, num_subcores=NS)
    f = functools.partial(
        pl.kernel,
        mesh=mesh,
        compiler_params=pltpu.CompilerParams(
            needs_layout_passes=False, use_tc_tiling_on_sc=False),
        out_type=[jax.ShapeDtypeStruct((OUT_DIM, NPAD), _f32),
                  jax.ShapeDtypeStruct((H, NPAD), _f32)],
        scratch_types=[
            pltpu.VMEM((CE,), jnp.int32),
            pltpu.VMEM((WROWS, CE), _f32),
            pltpu.VMEM((CE,), _f32),
            pltpu.VMEM((NPAD,), _f32),
            pltpu.VMEM((NPAD,), _f32),
            pltpu.VMEM((NPAD,), _f32),
            pltpu.VMEM((NPAD,), _f32),
            pltpu.VMEM((NPAD,), _f32),
        ],
    )(_scatter_body)
    return f(wT, eT, dst, z)


def _final_body(owt_ref, oet_ref, mexpt_ref, wo_ref, bo_ref, g_ref, b_ref,
                out_ref):
    numt = owt_ref[...]
    den = jnp.dot(mexpt_ref[...], oet_ref[...],
                  preferred_element_type=_f32)
    den = jnp.where(den == 0.0, 1.0, den)
    ratio = numt / den
    y = lax.dot_general(ratio, wo_ref[...], (((0,), (0,)), ((), ())),
                        preferred_element_type=_f32) + bo_ref[...]
    mu = jnp.mean(y, axis=1, keepdims=True)
    var = jnp.mean((y - mu) ** 2, axis=1, keepdims=True)
    out_ref[...] = (y - mu) * lax.rsqrt(var + EPS) * g_ref[...] + b_ref[...]


def _finalize(owT, oeT, MexpT, Wo, bo, gamma, beta):
    return pl.pallas_call(
        _final_body,
        grid=(pl.cdiv(N, NODE_BLK),),
        in_specs=[pl.BlockSpec((OUT_DIM, NODE_BLK), lambda i: (0, i)),
                  pl.BlockSpec((H, NODE_BLK), lambda i: (0, i)),
                  pl.BlockSpec((OUT_DIM, H), lambda i: (0, 0)),
                  pl.BlockSpec((OUT_DIM, OUT_DIM), lambda i: (0, 0)),
                  pl.BlockSpec((1, OUT_DIM), lambda i: (0, 0)),
                  pl.BlockSpec((1, OUT_DIM), lambda i: (0, 0)),
                  pl.BlockSpec((1, OUT_DIM), lambda i: (0, 0))],
        out_specs=pl.BlockSpec((NODE_BLK, OUT_DIM), lambda i: (i, 0)),
        out_shape=jax.ShapeDtypeStruct((N, OUT_DIM), _f32),
    )(owT, oeT, MexpT, Wo, bo.reshape(1, -1), gamma.reshape(1, -1),
      beta.reshape(1, -1))


def kernel(x, edge_index, edge_attr, batch, Wq, bq, Wk, bk, Wv, bv, We, be,
           Wo, bo, gamma, beta):
    src = edge_index[0]
    dst = edge_index[1]

    head_of = jnp.arange(OUT_DIM) // DH
    G = (head_of[:, None] == head_of[None, :]).astype(_f32) * SCALE
    Mexp = (jnp.arange(H)[:, None] == head_of[None, :]).astype(_f32)
    WeX = We @ Mexp
    beX = (be @ Mexp).reshape(1, OUT_DIM)
    Dsel = (jnp.arange(H)[:, None] * DH == jnp.arange(OUT_DIM)[None, :])
    Dsel = Dsel.astype(_f32)
    Eye = jnp.eye(OUT_DIM, dtype=_f32)
    MexpT = Mexp.T

    q, k, v = _qkv(x, Wq, Wk, Wv, bq, bk, bv)
    qd, ks, vs = _gather(src, dst, q, k, v)
    wT, eT = _edge_compute(qd, ks, vs, edge_attr, G, WeX, beX, Dsel, Eye)
    z = jnp.zeros((NPAD,), _f32)
    owT, oeT = _scatter(wT, eT, dst, z)
    out = _finalize(owT, oeT, MexpT, Wo, bo, gamma, beta)
    return out


# speedup vs baseline: 25.9247x; 1.1158x over previous
import functools

import jax
import jax.numpy as jnp
from jax import lax
from jax.experimental import pallas as pl
from jax.experimental.pallas import tpu as pltpu
from jax.experimental.pallas import tpu_sc as plsc

N = 10000
E = 320000
IN_DIM = 128
OUT_DIM = 128
EDGE_DIM = 16
H = 8
DH = OUT_DIM // H
SCALE = DH ** -0.5
EPS = 1e-5

NC = 2
NS = 16
NW = NC * NS
NPAD = 10240
WROWS = OUT_DIM // NW

CE = 8000
NCH = E // CE
GRP = CE // 16

QKV_BLK = 2000
EDGE_BLK = 2560
NODE_BLK = 2048

_f32 = jnp.float32


def _qkv_body(x_ref, wq_ref, wk_ref, wv_ref, bq_ref, bk_ref, bv_ref,
              q_ref, k_ref, v_ref):
    xb = x_ref[...]
    q_ref[...] = jnp.dot(xb, wq_ref[...], preferred_element_type=_f32) + bq_ref[...]
    k_ref[...] = jnp.dot(xb, wk_ref[...], preferred_element_type=_f32) + bk_ref[...]
    v_ref[...] = jnp.dot(xb, wv_ref[...], preferred_element_type=_f32) + bv_ref[...]


def _qkv(x, Wq, Wk, Wv, bq, bk, bv):
    blk = pl.BlockSpec((QKV_BLK, IN_DIM), lambda i: (i, 0))
    wspec = pl.BlockSpec((IN_DIM, OUT_DIM), lambda i: (0, 0))
    bspec = pl.BlockSpec((1, OUT_DIM), lambda i: (0, 0))
    return pl.pallas_call(
        _qkv_body,
        grid=(N // QKV_BLK,),
        in_specs=[blk, wspec, wspec, wspec, bspec, bspec, bspec],
        out_specs=[pl.BlockSpec((QKV_BLK, OUT_DIM), lambda i: (i, 0))] * 3,
        out_shape=[jax.ShapeDtypeStruct((N, OUT_DIM), _f32)] * 3,
    )(x, Wq, Wk, Wv, bq.reshape(1, -1), bk.reshape(1, -1), bv.reshape(1, -1))


GCH = 128
ET = E // NW
GFULL = ET // GCH
GTAIL = ET - GFULL * GCH


def _gather_body(src_hbm, dst_hbm, q_hbm, k_hbm, v_hbm,
                 qd_hbm, ks_hbm, vs_hbm,
                 sidx, didx, sidx_t, didx_t,
                 qv, kv, vv, qv_t, kv_t, vv_t,
                 sem0, sem1, sem2):
    cid = lax.axis_index("c")
    sid = lax.axis_index("s")
    wid = sid * NC + cid
    base = wid * ET

    def chunk(j, carry):
        off = pl.multiple_of(base + j * GCH, 8)
        pltpu.sync_copy(src_hbm.at[pl.ds(off, GCH)], sidx)
        pltpu.sync_copy(dst_hbm.at[pl.ds(off, GCH)], didx)
        cq = pltpu.async_copy(q_hbm.at[didx], qv, sem0)
        ck = pltpu.async_copy(k_hbm.at[sidx], kv, sem1)
        cv = pltpu.async_copy(v_hbm.at[sidx], vv, sem2)
        cq.wait()
        ck.wait()
        cv.wait()
        pltpu.sync_copy(qv, qd_hbm.at[pl.ds(off, GCH)])
        pltpu.sync_copy(kv, ks_hbm.at[pl.ds(off, GCH)])
        pltpu.sync_copy(vv, vs_hbm.at[pl.ds(off, GCH)])
        return carry

    lax.fori_loop(0, GFULL, chunk, 0)
    off = base + GFULL * GCH
    pltpu.sync_copy(src_hbm.at[pl.ds(off, GTAIL)], sidx_t)
    pltpu.sync_copy(dst_hbm.at[pl.ds(off, GTAIL)], didx_t)
    cq = pltpu.async_copy(q_hbm.at[didx_t], qv_t, sem0)
    ck = pltpu.async_copy(k_hbm.at[sidx_t], kv_t, sem1)
    cv = pltpu.async_copy(v_hbm.at[sidx_t], vv_t, sem2)
    cq.wait()
    ck.wait()
    cv.wait()
    pltpu.sync_copy(qv_t, qd_hbm.at[pl.ds(off, GTAIL)])
    pltpu.sync_copy(kv_t, ks_hbm.at[pl.ds(off, GTAIL)])
    pltpu.sync_copy(vv_t, vs_hbm.at[pl.ds(off, GTAIL)])


def _gather(src, dst, q, k, v):
    mesh = plsc.VectorSubcoreMesh(core_axis_name="c", subcore_axis_name="s",
                                  num_cores=NC, num_subcores=NS)
    f = functools.partial(
        pl.kernel,
        mesh=mesh,
        out_type=[jax.ShapeDtypeStruct((E, OUT_DIM), _f32)] * 3,
        scratch_types=[
            pltpu.VMEM((GCH,), jnp.int32),
            pltpu.VMEM((GCH,), jnp.int32),
            pltpu.VMEM((GTAIL,), jnp.int32),
            pltpu.VMEM((GTAIL,), jnp.int32),
            pltpu.VMEM((GCH, OUT_DIM), _f32),
            pltpu.VMEM((GCH, OUT_DIM), _f32),
            pltpu.VMEM((GCH, OUT_DIM), _f32),
            pltpu.VMEM((GTAIL, OUT_DIM), _f32),
            pltpu.VMEM((GTAIL, OUT_DIM), _f32),
            pltpu.VMEM((GTAIL, OUT_DIM), _f32),
            pltpu.SemaphoreType.DMA,
            pltpu.SemaphoreType.DMA,
            pltpu.SemaphoreType.DMA,
        ],
    )(_gather_body)
    return f(src, dst, q, k, v)


def _edge_body(qd_ref, ks_ref, vs_ref, ea_ref, g_ref, wex_ref, bex_ref,
               dsel_ref, eye_ref, wt_ref, et_ref):
    prod = qd_ref[...] * ks_ref[...]
    score = (jnp.dot(prod, g_ref[...], preferred_element_type=_f32)
             + jnp.dot(ea_ref[...], wex_ref[...], preferred_element_type=_f32)
             + bex_ref[...])
    exx = jnp.exp(score)
    w = vs_ref[...] * exx
    wt_ref[...] = lax.dot_general(
        eye_ref[...], w, (((0,), (1,)), ((), ())),
        preferred_element_type=_f32)
    et_ref[...] = lax.dot_general(
        dsel_ref[...], exx, (((1,), (1,)), ((), ())),
        preferred_element_type=_f32)


def _edge_compute(qd, ks, vs, ea, G, WeX, beX, Dsel, Eye):
    eblk = pl.BlockSpec((EDGE_BLK, OUT_DIM), lambda i: (i, 0))
    return pl.pallas_call(
        _edge_body,
        grid=(E // EDGE_BLK,),
        in_specs=[eblk, eblk, eblk,
                  pl.BlockSpec((EDGE_BLK, EDGE_DIM), lambda i: (i, 0)),
                  pl.BlockSpec((OUT_DIM, OUT_DIM), lambda i: (0, 0)),
                  pl.BlockSpec((EDGE_DIM, OUT_DIM), lambda i: (0, 0)),
                  pl.BlockSpec((1, OUT_DIM), lambda i: (0, 0)),
                  pl.BlockSpec((H, OUT_DIM), lambda i: (0, 0)),
                  pl.BlockSpec((OUT_DIM, OUT_DIM), lambda i: (0, 0))],
        out_specs=[pl.BlockSpec((OUT_DIM, EDGE_BLK), lambda i: (0, i)),
                   pl.BlockSpec((H, EDGE_BLK), lambda i: (0, i))],
        out_shape=[jax.ShapeDtypeStruct((OUT_DIM, E), _f32),
                   jax.ShapeDtypeStruct((H, E), _f32)],
    )(qd, ks, vs, ea, G, WeX, beX, Dsel, Eye)


def _scatter_body(wt_hbm, et_hbm, dst_hbm, z_hbm, owt_hbm, oet_hbm,
                  didx, wb, eb, a0, a1, a2, a3, ae):
    cid = lax.axis_index("c")
    sid = lax.axis_index("s")
    u = sid * NC + cid
    accs = [a0, a1, a2, a3]
    for a in accs:
        pltpu.sync_copy(z_hbm, a)
    pltpu.sync_copy(z_hbm, ae)
    row0 = u * WROWS
    is_e = u < H

    def chunk(j, carry):
        off = pl.multiple_of(j * CE, 8)
        pltpu.sync_copy(dst_hbm.at[pl.ds(off, CE)], didx)
        for r in range(WROWS):
            pltpu.sync_copy(wt_hbm.at[row0 + r, pl.ds(off, CE)], wb.at[r])

        @pl.when(is_e)
        def _():
            pltpu.sync_copy(et_hbm.at[u, pl.ds(off, CE)], eb)

        def grp(g, c2):
            s16 = pl.multiple_of(g * 16, 16)
            idxv = didx[pl.ds(s16, 16)]
            for r in range(WROWS):
                plsc.addupdate_scatter(accs[r], [idxv],
                                       wb[r, pl.ds(s16, 16)])
            return c2

        lax.fori_loop(0, GRP, grp, 0, unroll=4)

        @pl.when(is_e)
        def _():
            def grp_e(g, c2):
                s16 = pl.multiple_of(g * 16, 16)
                idxv = didx[pl.ds(s16, 16)]
                plsc.addupdate_scatter(ae, [idxv], eb[pl.ds(s16, 16)])
                return c2
            lax.fori_loop(0, GRP, grp_e, 0, unroll=4)
        return carry

    lax.fori_loop(0, NCH, chunk, 0)
    for r in range(WROWS):
        pltpu.sync_copy(accs[r], owt_hbm.at[row0 + r])

    @pl.when(is_e)
    def _():
        pltpu.sync_copy(ae, oet_hbm.at[u])


def _scatter(wT, eT, dst, z):
    mesh = plsc.VectorSubcoreMesh(core_axis_name="c", subcore_axis_name="s",
                                  num_cores=NC, num_subcores=NS)
    f = functools.partial(
        pl.kernel,
        mesh=mesh,
        compiler_params=pltpu.CompilerParams(
            needs_layout_passes=False, use_tc_tiling_on_sc=False),
        out_type=[jax.ShapeDtypeStruct((OUT_DIM, NPAD), _f32),
                  jax.ShapeDtypeStruct((H, NPAD), _f32)],
        scratch_types=[
            pltpu.VMEM((CE,), jnp.int32),
            pltpu.VMEM((WROWS, CE), _f32),
            pltpu.VMEM((CE,), _f32),
            pltpu.VMEM((NPAD,), _f32),
            pltpu.VMEM((NPAD,), _f32),
            pltpu.VMEM((NPAD,), _f32),
            pltpu.VMEM((NPAD,), _f32),
            pltpu.VMEM((NPAD,), _f32),
        ],
    )(_scatter_body)
    return f(wT, eT, dst, z)


def _final_body(owt_ref, oet_ref, mexpt_ref, wo_ref, bo_ref, g_ref, b_ref,
                out_ref):
    numt = owt_ref[...]
    den = jnp.dot(mexpt_ref[...], oet_ref[...],
                  preferred_element_type=_f32)
    den = jnp.where(den == 0.0, 1.0, den)
    ratio = numt / den
    y = lax.dot_general(ratio, wo_ref[...], (((0,), (0,)), ((), ())),
                        preferred_element_type=_f32) + bo_ref[...]
    mu = jnp.mean(y, axis=1, keepdims=True)
    var = jnp.mean((y - mu) ** 2, axis=1, keepdims=True)
    out_ref[...] = (y - mu) * lax.rsqrt(var + EPS) * g_ref[...] + b_ref[...]


def _finalize(owT, oeT, MexpT, Wo, bo, gamma, beta):
    return pl.pallas_call(
        _final_body,
        grid=(pl.cdiv(N, NODE_BLK),),
        in_specs=[pl.BlockSpec((OUT_DIM, NODE_BLK), lambda i: (0, i)),
                  pl.BlockSpec((H, NODE_BLK), lambda i: (0, i)),
                  pl.BlockSpec((OUT_DIM, H), lambda i: (0, 0)),
                  pl.BlockSpec((OUT_DIM, OUT_DIM), lambda i: (0, 0)),
                  pl.BlockSpec((1, OUT_DIM), lambda i: (0, 0)),
                  pl.BlockSpec((1, OUT_DIM), lambda i: (0, 0)),
                  pl.BlockSpec((1, OUT_DIM), lambda i: (0, 0))],
        out_specs=pl.BlockSpec((NODE_BLK, OUT_DIM), lambda i: (i, 0)),
        out_shape=jax.ShapeDtypeStruct((N, OUT_DIM), _f32),
    )(owT, oeT, MexpT, Wo, bo.reshape(1, -1), gamma.reshape(1, -1),
      beta.reshape(1, -1))


def kernel(x, edge_index, edge_attr, batch, Wq, bq, Wk, bk, Wv, bv, We, be,
           Wo, bo, gamma, beta):
    src = edge_index[0]
    dst = edge_index[1]

    head_of = jnp.arange(OUT_DIM) // DH
    G = (head_of[:, None] == head_of[None, :]).astype(_f32) * SCALE
    Mexp = (jnp.arange(H)[:, None] == head_of[None, :]).astype(_f32)
    WeX = We @ Mexp
    beX = (be @ Mexp).reshape(1, OUT_DIM)
    Dsel = (jnp.arange(H)[:, None] * DH == jnp.arange(OUT_DIM)[None, :])
    Dsel = Dsel.astype(_f32)
    Eye = jnp.eye(OUT_DIM, dtype=_f32)
    MexpT = Mexp.T

    q, k, v = _qkv(x, Wq, Wk, Wv, bq, bk, bv)
    qd, ks, vs = _gather(src, dst, q, k, v)
    wT, eT = _edge_compute(qd, ks, vs, edge_attr, G, WeX, beX, Dsel, Eye)
    z = jnp.zeros((NPAD,), _f32)
    owT, oeT = _scatter(wT, eT, dst, z)
    out = _finalize(owT, oeT, MexpT, Wo, bo, gamma, beta)
    return out


# speedup vs baseline: 26.8047x; 1.0339x over previous
import functools

import jax
import jax.numpy as jnp
from jax import lax
from jax.experimental import pallas as pl
from jax.experimental.pallas import tpu as pltpu
from jax.experimental.pallas import tpu_sc as plsc

N = 10000
E = 320000
IN_DIM = 128
OUT_DIM = 128
EDGE_DIM = 16
H = 8
DH = OUT_DIM // H
SCALE = DH ** -0.5
EPS = 1e-5

NC = 2
NS = 16
NW = NC * NS
NPAD = 10240
WROWS = OUT_DIM // NW

CE = 8000
NCH = E // CE
GRP = CE // 16

QKV_BLK = 2000
EDGE_BLK = 2560
NODE_BLK = 2048

_f32 = jnp.float32


def _qkv_body(x_ref, wq_ref, wk_ref, wv_ref, bq_ref, bk_ref, bv_ref,
              q_ref, k_ref, v_ref):
    xb = x_ref[...]
    q_ref[...] = jnp.dot(xb, wq_ref[...], preferred_element_type=_f32) + bq_ref[...]
    k_ref[...] = jnp.dot(xb, wk_ref[...], preferred_element_type=_f32) + bk_ref[...]
    v_ref[...] = jnp.dot(xb, wv_ref[...], preferred_element_type=_f32) + bv_ref[...]


def _qkv(x, Wq, Wk, Wv, bq, bk, bv):
    blk = pl.BlockSpec((QKV_BLK, IN_DIM), lambda i: (i, 0))
    wspec = pl.BlockSpec((IN_DIM, OUT_DIM), lambda i: (0, 0))
    bspec = pl.BlockSpec((1, OUT_DIM), lambda i: (0, 0))
    return pl.pallas_call(
        _qkv_body,
        grid=(N // QKV_BLK,),
        in_specs=[blk, wspec, wspec, wspec, bspec, bspec, bspec],
        out_specs=[pl.BlockSpec((QKV_BLK, OUT_DIM), lambda i: (i, 0))] * 3,
        out_shape=[jax.ShapeDtypeStruct((N, OUT_DIM), _f32)] * 3,
    )(x, Wq, Wk, Wv, bq.reshape(1, -1), bk.reshape(1, -1), bv.reshape(1, -1))


GCH = 128
ET = E // NW
GFULL = ET // GCH
GTAIL = ET - GFULL * GCH


def _gather_body(src_hbm, dst_hbm, q_hbm, k_hbm, v_hbm,
                 qd_hbm, ks_hbm, vs_hbm,
                 sidx2, didx2, sidx_t, didx_t,
                 qv2, kv2, vv2, qv_t, kv_t, vv_t,
                 gsem, wsem, sem2):
    cid = lax.axis_index("c")
    sid = lax.axis_index("s")
    wid = sid * NC + cid
    base = wid * ET

    def fetch(j, par):
        off = pl.multiple_of(base + j * GCH, 8)
        pltpu.sync_copy(src_hbm.at[pl.ds(off, GCH)], sidx2.at[par])
        pltpu.sync_copy(dst_hbm.at[pl.ds(off, GCH)], didx2.at[par])
        pltpu.async_copy(q_hbm.at[didx2.at[par]], qv2.at[par], gsem)
        pltpu.async_copy(k_hbm.at[sidx2.at[par]], kv2.at[par], gsem)
        pltpu.async_copy(v_hbm.at[sidx2.at[par]], vv2.at[par], gsem)

    fetch(0, 0)

    def chunk(j, carry):
        par = lax.rem(j, 2)
        off = pl.multiple_of(base + j * GCH, 8)

        @pl.when(j > 0)
        def _():
            offp = pl.multiple_of(off - GCH, 8)
            pltpu.make_async_copy(qv2.at[1 - par],
                                  qd_hbm.at[pl.ds(offp, GCH)], wsem).wait()
            pltpu.make_async_copy(kv2.at[1 - par],
                                  ks_hbm.at[pl.ds(offp, GCH)], wsem).wait()
            pltpu.make_async_copy(vv2.at[1 - par],
                                  vs_hbm.at[pl.ds(offp, GCH)], wsem).wait()

        pltpu.make_async_copy(q_hbm.at[didx2.at[par]], qv2.at[par],
                              gsem).wait()
        pltpu.make_async_copy(k_hbm.at[sidx2.at[par]], kv2.at[par],
                              gsem).wait()
        pltpu.make_async_copy(v_hbm.at[sidx2.at[par]], vv2.at[par],
                              gsem).wait()

        @pl.when(j + 1 < GFULL)
        def _():
            fetch(j + 1, 1 - par)

        pltpu.async_copy(qv2.at[par], qd_hbm.at[pl.ds(off, GCH)], wsem)
        pltpu.async_copy(kv2.at[par], ks_hbm.at[pl.ds(off, GCH)], wsem)
        pltpu.async_copy(vv2.at[par], vs_hbm.at[pl.ds(off, GCH)], wsem)
        return carry

    lax.fori_loop(0, GFULL, chunk, 0)
    lastpar = (GFULL - 1) % 2
    lastoff = base + (GFULL - 1) * GCH
    pltpu.make_async_copy(qv2.at[lastpar],
                          qd_hbm.at[pl.ds(lastoff, GCH)], wsem).wait()
    pltpu.make_async_copy(kv2.at[lastpar],
                          ks_hbm.at[pl.ds(lastoff, GCH)], wsem).wait()
    pltpu.make_async_copy(vv2.at[lastpar],
                          vs_hbm.at[pl.ds(lastoff, GCH)], wsem).wait()
    off = base + GFULL * GCH
    pltpu.sync_copy(src_hbm.at[pl.ds(off, GTAIL)], sidx_t)
    pltpu.sync_copy(dst_hbm.at[pl.ds(off, GTAIL)], didx_t)
    cq = pltpu.async_copy(q_hbm.at[didx_t], qv_t, gsem)
    ck = pltpu.async_copy(k_hbm.at[sidx_t], kv_t, wsem)
    cv = pltpu.async_copy(v_hbm.at[sidx_t], vv_t, sem2)
    cq.wait()
    ck.wait()
    cv.wait()
    pltpu.sync_copy(qv_t, qd_hbm.at[pl.ds(off, GTAIL)])
    pltpu.sync_copy(kv_t, ks_hbm.at[pl.ds(off, GTAIL)])
    pltpu.sync_copy(vv_t, vs_hbm.at[pl.ds(off, GTAIL)])


def _gather(src, dst, q, k, v):
    mesh = plsc.VectorSubcoreMesh(core_axis_name="c", subcore_axis_name="s",
                                  num_cores=NC, num_subcores=NS)
    f = functools.partial(
        pl.kernel,
        mesh=mesh,
        out_type=[jax.ShapeDtypeStruct((E, OUT_DIM), _f32)] * 3,
        scratch_types=[
            pltpu.VMEM((2, GCH), jnp.int32),
            pltpu.VMEM((2, GCH), jnp.int32),
            pltpu.VMEM((GTAIL,), jnp.int32),
            pltpu.VMEM((GTAIL,), jnp.int32),
            pltpu.VMEM((2, GCH, OUT_DIM), _f32),
            pltpu.VMEM((2, GCH, OUT_DIM), _f32),
            pltpu.VMEM((2, GCH, OUT_DIM), _f32),
            pltpu.VMEM((GTAIL, OUT_DIM), _f32),
            pltpu.VMEM((GTAIL, OUT_DIM), _f32),
            pltpu.VMEM((GTAIL, OUT_DIM), _f32),
            pltpu.SemaphoreType.DMA,
            pltpu.SemaphoreType.DMA,
            pltpu.SemaphoreType.DMA,
        ],
    )(_gather_body)
    return f(src, dst, q, k, v)


def _edge_body(qd_ref, ks_ref, vs_ref, ea_ref, g_ref, wex_ref, bex_ref,
               dsel_ref, eye_ref, wt_ref, et_ref):
    prod = qd_ref[...] * ks_ref[...]
    score = (jnp.dot(prod, g_ref[...], preferred_element_type=_f32)
             + jnp.dot(ea_ref[...], wex_ref[...], preferred_element_type=_f32)
             + bex_ref[...])
    exx = jnp.exp(score)
    w = vs_ref[...] * exx
    wt_ref[...] = lax.dot_general(
        eye_ref[...], w, (((0,), (1,)), ((), ())),
        preferred_element_type=_f32)
    et_ref[...] = lax.dot_general(
        dsel_ref[...], exx, (((1,), (1,)), ((), ())),
        preferred_element_type=_f32)


def _edge_compute(qd, ks, vs, ea, G, WeX, beX, Dsel, Eye):
    eblk = pl.BlockSpec((EDGE_BLK, OUT_DIM), lambda i: (i, 0))
    return pl.pallas_call(
        _edge_body,
        grid=(E // EDGE_BLK,),
        in_specs=[eblk, eblk, eblk,
                  pl.BlockSpec((EDGE_BLK, EDGE_DIM), lambda i: (i, 0)),
                  pl.BlockSpec((OUT_DIM, OUT_DIM), lambda i: (0, 0)),
                  pl.BlockSpec((EDGE_DIM, OUT_DIM), lambda i: (0, 0)),
                  pl.BlockSpec((1, OUT_DIM), lambda i: (0, 0)),
                  pl.BlockSpec((H, OUT_DIM), lambda i: (0, 0)),
                  pl.BlockSpec((OUT_DIM, OUT_DIM), lambda i: (0, 0))],
        out_specs=[pl.BlockSpec((OUT_DIM, EDGE_BLK), lambda i: (0, i)),
                   pl.BlockSpec((H, EDGE_BLK), lambda i: (0, i))],
        out_shape=[jax.ShapeDtypeStruct((OUT_DIM, E), _f32),
                   jax.ShapeDtypeStruct((H, E), _f32)],
    )(qd, ks, vs, ea, G, WeX, beX, Dsel, Eye)


def _scatter_body(wt_hbm, et_hbm, dst_hbm, z_hbm, owt_hbm, oet_hbm,
                  didx, wb, eb, a0, a1, a2, a3, ae):
    cid = lax.axis_index("c")
    sid = lax.axis_index("s")
    u = sid * NC + cid
    accs = [a0, a1, a2, a3]
    for a in accs:
        pltpu.sync_copy(z_hbm, a)
    pltpu.sync_copy(z_hbm, ae)
    row0 = u * WROWS
    is_e = u < H

    def chunk(j, carry):
        off = pl.multiple_of(j * CE, 8)
        pltpu.sync_copy(dst_hbm.at[pl.ds(off, CE)], didx)
        for r in range(WROWS):
            pltpu.sync_copy(wt_hbm.at[row0 + r, pl.ds(off, CE)], wb.at[r])

        @pl.when(is_e)
        def _():
            pltpu.sync_copy(et_hbm.at[u, pl.ds(off, CE)], eb)

        def grp(g, c2):
            s16 = pl.multiple_of(g * 16, 16)
            idxv = didx[pl.ds(s16, 16)]
            for r in range(WROWS):
                plsc.addupdate_scatter(accs[r], [idxv],
                                       wb[r, pl.ds(s16, 16)])
            return c2

        lax.fori_loop(0, GRP, grp, 0, unroll=4)

        @pl.when(is_e)
        def _():
            def grp_e(g, c2):
                s16 = pl.multiple_of(g * 16, 16)
                idxv = didx[pl.ds(s16, 16)]
                plsc.addupdate_scatter(ae, [idxv], eb[pl.ds(s16, 16)])
                return c2
            lax.fori_loop(0, GRP, grp_e, 0, unroll=4)
        return carry

    lax.fori_loop(0, NCH, chunk, 0)
    for r in range(WROWS):
        pltpu.sync_copy(accs[r], owt_hbm.at[row0 + r])

    @pl.when(is_e)
    def _():
        pltpu.sync_copy(ae, oet_hbm.at[u])


def _scatter(wT, eT, dst, z):
    mesh = plsc.VectorSubcoreMesh(core_axis_name="c", subcore_axis_name="s",
                                  num_cores=NC, num_subcores=NS)
    f = functools.partial(
        pl.kernel,
        mesh=mesh,
        compiler_params=pltpu.CompilerParams(
            needs_layout_passes=False, use_tc_tiling_on_sc=False),
        out_type=[jax.ShapeDtypeStruct((OUT_DIM, NPAD), _f32),
                  jax.ShapeDtypeStruct((H, NPAD), _f32)],
        scratch_types=[
            pltpu.VMEM((CE,), jnp.int32),
            pltpu.VMEM((WROWS, CE), _f32),
            pltpu.VMEM((CE,), _f32),
            pltpu.VMEM((NPAD,), _f32),
            pltpu.VMEM((NPAD,), _f32),
            pltpu.VMEM((NPAD,), _f32),
            pltpu.VMEM((NPAD,), _f32),
            pltpu.VMEM((NPAD,), _f32),
        ],
    )(_scatter_body)
    return f(wT, eT, dst, z)


def _final_body(owt_ref, oet_ref, mexpt_ref, wo_ref, bo_ref, g_ref, b_ref,
                out_ref):
    numt = owt_ref[...]
    den = jnp.dot(mexpt_ref[...], oet_ref[...],
                  preferred_element_type=_f32)
    den = jnp.where(den == 0.0, 1.0, den)
    ratio = numt / den
    y = lax.dot_general(ratio, wo_ref[...], (((0,), (0,)), ((), ())),
                        preferred_element_type=_f32) + bo_ref[...]
    mu = jnp.mean(y, axis=1, keepdims=True)
    var = jnp.mean((y - mu) ** 2, axis=1, keepdims=True)
    out_ref[...] = (y - mu) * lax.rsqrt(var + EPS) * g_ref[...] + b_ref[...]


def _finalize(owT, oeT, MexpT, Wo, bo, gamma, beta):
    return pl.pallas_call(
        _final_body,
        grid=(pl.cdiv(N, NODE_BLK),),
        in_specs=[pl.BlockSpec((OUT_DIM, NODE_BLK), lambda i: (0, i)),
                  pl.BlockSpec((H, NODE_BLK), lambda i: (0, i)),
                  pl.BlockSpec((OUT_DIM, H), lambda i: (0, 0)),
                  pl.BlockSpec((OUT_DIM, OUT_DIM), lambda i: (0, 0)),
                  pl.BlockSpec((1, OUT_DIM), lambda i: (0, 0)),
                  pl.BlockSpec((1, OUT_DIM), lambda i: (0, 0)),
                  pl.BlockSpec((1, OUT_DIM), lambda i: (0, 0))],
        out_specs=pl.BlockSpec((NODE_BLK, OUT_DIM), lambda i: (i, 0)),
        out_shape=jax.ShapeDtypeStruct((N, OUT_DIM), _f32),
    )(owT, oeT, MexpT, Wo, bo.reshape(1, -1), gamma.reshape(1, -1),
      beta.reshape(1, -1))


def kernel(x, edge_index, edge_attr, batch, Wq, bq, Wk, bk, Wv, bv, We, be,
           Wo, bo, gamma, beta):
    src = edge_index[0]
    dst = edge_index[1]

    head_of = jnp.arange(OUT_DIM) // DH
    G = (head_of[:, None] == head_of[None, :]).astype(_f32) * SCALE
    Mexp = (jnp.arange(H)[:, None] == head_of[None, :]).astype(_f32)
    WeX = We @ Mexp
    beX = (be @ Mexp).reshape(1, OUT_DIM)
    Dsel = (jnp.arange(H)[:, None] * DH == jnp.arange(OUT_DIM)[None, :])
    Dsel = Dsel.astype(_f32)
    Eye = jnp.eye(OUT_DIM, dtype=_f32)
    MexpT = Mexp.T

    q, k, v = _qkv(x, Wq, Wk, Wv, bq, bk, bv)
    qd, ks, vs = _gather(src, dst, q, k, v)
    wT, eT = _edge_compute(qd, ks, vs, edge_attr, G, WeX, beX, Dsel, Eye)
    z = jnp.zeros((NPAD,), _f32)
    owT, oeT = _scatter(wT, eT, dst, z)
    out = _finalize(owT, oeT, MexpT, Wo, bo, gamma, beta)
    return out
